# baseline (speedup 1.0000x reference)
import functools

import jax
import jax.numpy as jnp
from jax import lax
from jax.experimental import pallas as pl
from jax.experimental.pallas import tpu as pltpu
from jax.experimental.pallas import tpu_sc as plsc

NC = 2
NS = 16
NW = NC * NS
L = 16
CHUNK = 128


def _mirt_body(nchunk, nvec,
               stu_hbm, exer_hbm, theta_hbm, alpha_hbm, beta_hbm, out_hbm,
               stu_v, exer_v, t0i, t1i, a0i, a1i,
               t0_v, t1_v, a0_v, a1_v, b_v, out_v, sem):
    wid = lax.axis_index("s") * NC + lax.axis_index("c")

    pltpu.sync_copy(stu_hbm.at[wid], stu_v)
    pltpu.sync_copy(exer_hbm.at[wid], exer_v)

    def idx_body(j, carry):
        sl = pl.ds(j * L, L)
        s2 = stu_v[sl] * 2
        t0i[sl] = s2
        t1i[sl] = s2 + 1
        e2 = exer_v[sl] * 2
        a0i[sl] = e2
        a1i[sl] = e2 + 1
        return carry

    lax.fori_loop(0, nvec, idx_body, 0)

    copies = []
    for c in range(nchunk):
        sl = pl.ds(c * CHUNK, CHUNK)
        copies.append(pltpu.async_copy(theta_hbm.at[t0i.at[sl]], t0_v.at[sl], sem))
        copies.append(pltpu.async_copy(theta_hbm.at[t1i.at[sl]], t1_v.at[sl], sem))
        copies.append(pltpu.async_copy(alpha_hbm.at[a0i.at[sl]], a0_v.at[sl], sem))
        copies.append(pltpu.async_copy(alpha_hbm.at[a1i.at[sl]], a1_v.at[sl], sem))
        copies.append(pltpu.async_copy(beta_hbm.at[exer_v.at[sl]], b_v.at[sl], sem))
    for cp in copies:
        cp.wait()

    def vec_body(j, carry):
        sl = pl.ds(j * L, L)
        t0 = t0_v[sl]
        t1 = t1_v[sl]
        a0 = a0_v[sl]
        a1 = a1_v[sl]
        b = b_v[sl]
        pred = a0 * (t0 - b) + a1 * (t1 - b)
        out_v[sl] = 1.0 / (1.0 + jnp.exp(-pred))
        return carry

    lax.fori_loop(0, nvec, vec_body, 0)

    pltpu.sync_copy(out_v, out_hbm.at[wid])


def _build(batch):
    bpw = batch // NW
    nchunk = bpw // CHUNK
    nvec = bpw // L
    mesh = plsc.VectorSubcoreMesh(core_axis_name="c", subcore_axis_name="s")
    idx = pltpu.VMEM((bpw,), jnp.int32)
    val = pltpu.VMEM((bpw,), jnp.float32)
    return functools.partial(
        pl.kernel,
        out_type=jax.ShapeDtypeStruct((NW, bpw), jnp.float32),
        mesh=mesh,
        scratch_types=[idx, idx, idx, idx, idx, idx,
                       val, val, val, val, val, val,
                       pltpu.SemaphoreType.DMA],
    )(functools.partial(_mirt_body, nchunk, nvec))


def kernel(stu_id, exer_id, theta_table, alpha_table, beta_table):
    batch = stu_id.shape[0]
    bpw = batch // NW
    stu = stu_id.astype(jnp.int32).reshape(NW, bpw)
    exer = exer_id.astype(jnp.int32).reshape(NW, bpw)
    theta_flat = theta_table.reshape(-1)
    alpha_flat = alpha_table.reshape(-1)
    beta_flat = beta_table.reshape(-1)
    out = _build(batch)(stu, exer, theta_flat, alpha_flat, beta_flat)
    return out.reshape(batch)


# speedup vs baseline: 17.0966x; 17.0966x over previous
import functools

import jax
import jax.numpy as jnp
from jax import lax
from jax.experimental import pallas as pl
from jax.experimental.pallas import tpu as pltpu
from jax.experimental.pallas import tpu_sc as plsc

NC = 2
NS = 16
NW = NC * NS
L = 16
CHUNK = 128


def _mirt_body(nchunk, nvec,
               stu_hbm, exer_hbm, t0_hbm, t1_hbm, a0_hbm, a1_hbm, b_hbm,
               out_hbm,
               stu_v, exer_v, t0_v, t1_v, a0_v, a1_v, b_v, out_v, sem):
    wid = lax.axis_index("s") * NC + lax.axis_index("c")

    pltpu.sync_copy(stu_hbm.at[wid], stu_v)
    pltpu.sync_copy(exer_hbm.at[wid], exer_v)

    copies = []
    for c in range(nchunk):
        sl = pl.ds(c * CHUNK, CHUNK)
        copies.append(pltpu.async_copy(t0_hbm.at[stu_v.at[sl]], t0_v.at[sl], sem))
        copies.append(pltpu.async_copy(t1_hbm.at[stu_v.at[sl]], t1_v.at[sl], sem))
        copies.append(pltpu.async_copy(a0_hbm.at[exer_v.at[sl]], a0_v.at[sl], sem))
        copies.append(pltpu.async_copy(a1_hbm.at[exer_v.at[sl]], a1_v.at[sl], sem))
        copies.append(pltpu.async_copy(b_hbm.at[exer_v.at[sl]], b_v.at[sl], sem))
    for cp in copies:
        cp.wait()

    def vec_body(j, carry):
        sl = pl.ds(j * L, L)
        t0 = t0_v[sl]
        t1 = t1_v[sl]
        a0 = a0_v[sl]
        a1 = a1_v[sl]
        b = b_v[sl]
        pred = a0 * (t0 - b) + a1 * (t1 - b)
        out_v[sl] = 1.0 / (1.0 + jnp.exp(-pred))
        return carry

    lax.fori_loop(0, nvec, vec_body, 0)

    pltpu.sync_copy(out_v, out_hbm.at[wid])


def _build(batch):
    bpw = batch // NW
    nchunk = bpw // CHUNK
    nvec = bpw // L
    mesh = plsc.VectorSubcoreMesh(core_axis_name="c", subcore_axis_name="s")
    idx = pltpu.VMEM((bpw,), jnp.int32)
    val = pltpu.VMEM((bpw,), jnp.float32)
    return functools.partial(
        pl.kernel,
        out_type=jax.ShapeDtypeStruct((NW, bpw), jnp.float32),
        mesh=mesh,
        scratch_types=[idx, idx,
                       val, val, val, val, val, val,
                       pltpu.SemaphoreType.DMA],
    )(functools.partial(_mirt_body, nchunk, nvec))


def kernel(stu_id, exer_id, theta_table, alpha_table, beta_table):
    batch = stu_id.shape[0]
    bpw = batch // NW
    stu = stu_id.astype(jnp.int32).reshape(NW, bpw)
    exer = exer_id.astype(jnp.int32).reshape(NW, bpw)
    t0 = theta_table[:, 0]
    t1 = theta_table[:, 1]
    a0 = alpha_table[:, 0]
    a1 = alpha_table[:, 1]
    b = beta_table[:, 0]
    out = _build(batch)(stu, exer, t0, t1, a0, a1, b)
    return out.reshape(batch)
